# Initial kernel scaffold; baseline (speedup 1.0000x reference)
import jax
import jax.numpy as jnp
from jax.experimental import pallas as pl


def kernel(x, means_idx_0, prop_means_idx_0, mean_fields):
    raise NotImplementedError("write your pallas kernel here")


# speedup vs baseline: 1.5048x; 1.5048x over previous
import functools

import jax
import jax.numpy as jnp
from jax.experimental import pallas as pl
from jax.experimental.pallas import tpu as pltpu

_LAM = 0.9
_S = 64
_B = 32
_E = 2 * _B
_BLOCK = 8192


def _combine_kernel(s_ref, a_ref, mf_ref, x_ref, out_ref, w_ref, c_ref):
    @pl.when(pl.program_id(0) == 0)
    def _():
        slot = jax.lax.broadcasted_iota(jnp.int32, (_S, _E), 0)
        s_ev = s_ref[0, :][None, :]
        a_ev = a_ref[0, :][None, :]
        hit = slot == s_ev
        logg = jnp.where(hit, jnp.log1p(-a_ev), 0.0)
        tr = jax.lax.broadcasted_iota(jnp.int32, (_E, _E), 0)
        tc = jax.lax.broadcasted_iota(jnp.int32, (_E, _E), 1)
        tri = (tr > tc).astype(jnp.float32)
        suff = jnp.exp(jax.lax.dot(logg, tri, preferred_element_type=jnp.float32))
        contrib = jnp.where(hit, a_ev * suff, 0.0)
        pt = jax.lax.broadcasted_iota(jnp.int32, (_E, _B), 0) // 2
        pb = jax.lax.broadcasted_iota(jnp.int32, (_E, _B), 1)
        fold = (pt == pb).astype(jnp.float32)
        w_ref[...] = jax.lax.dot(contrib, fold, preferred_element_type=jnp.float32)
        c_ref[...] = jnp.exp(jnp.sum(logg, axis=1, keepdims=True))

    out_ref[...] = c_ref[...] * mf_ref[...] + jax.lax.dot(
        w_ref[...], x_ref[...], preferred_element_type=jnp.float32)


def kernel(x, means_idx_0, prop_means_idx_0, mean_fields):
    b, ch, h, w = x.shape
    s = mean_fields.shape[0]
    n = ch * h * w
    xf = x.reshape(b, n)
    mf = mean_fields.reshape(s, n)
    i0 = means_idx_0.astype(jnp.int32)
    p0 = prop_means_idx_0.astype(jnp.float32)
    rate = jnp.float32(1.0 - _LAM)
    a_ev = jnp.stack([p0 * rate, (1.0 - p0) * rate], axis=1).reshape(1, 2 * b)
    s_ev = jnp.stack([i0, i0 + 1], axis=1).reshape(1, 2 * b)

    grid = (n // _BLOCK,)
    out = pl.pallas_call(
        _combine_kernel,
        grid=grid,
        in_specs=[
            pl.BlockSpec((1, _E), lambda j: (0, 0)),
            pl.BlockSpec((1, _E), lambda j: (0, 0)),
            pl.BlockSpec((s, _BLOCK), lambda j: (0, j)),
            pl.BlockSpec((b, _BLOCK), lambda j: (0, j)),
        ],
        out_specs=pl.BlockSpec((s, _BLOCK), lambda j: (0, j)),
        out_shape=jax.ShapeDtypeStruct((s, n), jnp.float32),
        scratch_shapes=[
            pltpu.VMEM((_S, _B), jnp.float32),
            pltpu.VMEM((_S, 1), jnp.float32),
        ],
    )(s_ev, a_ev, mf, xf)
    return out.reshape(s, ch, h, w)


# speedup vs baseline: 1.5256x; 1.0138x over previous
import functools

import jax
import jax.numpy as jnp
from jax.experimental import pallas as pl
from jax.experimental.pallas import tpu as pltpu

_LAM = 0.9
_S = 64
_B = 32
_E = 2 * _B
_BLOCK = 32768


def _combine_kernel(s_ref, a_ref, mf_ref, x_ref, out_ref, w_ref, c_ref):
    @pl.when(pl.program_id(0) == 0)
    def _():
        slot = jax.lax.broadcasted_iota(jnp.int32, (_S, _E), 0)
        s_ev = s_ref[0, :][None, :]
        a_ev = a_ref[0, :][None, :]
        hit = slot == s_ev
        logg = jnp.where(hit, jnp.log1p(-a_ev), 0.0)
        tr = jax.lax.broadcasted_iota(jnp.int32, (_E, _E), 0)
        tc = jax.lax.broadcasted_iota(jnp.int32, (_E, _E), 1)
        tri = (tr > tc).astype(jnp.float32)
        suff = jnp.exp(jax.lax.dot(logg, tri, preferred_element_type=jnp.float32))
        contrib = jnp.where(hit, a_ev * suff, 0.0)
        pt = jax.lax.broadcasted_iota(jnp.int32, (_E, _B), 0) // 2
        pb = jax.lax.broadcasted_iota(jnp.int32, (_E, _B), 1)
        fold = (pt == pb).astype(jnp.float32)
        w_ref[...] = jax.lax.dot(contrib, fold, preferred_element_type=jnp.float32)
        c_ref[...] = jnp.exp(jnp.sum(logg, axis=1, keepdims=True))

    out_ref[...] = c_ref[...] * mf_ref[...] + jax.lax.dot(
        w_ref[...], x_ref[...], preferred_element_type=jnp.float32)


def kernel(x, means_idx_0, prop_means_idx_0, mean_fields):
    b, ch, h, w = x.shape
    s = mean_fields.shape[0]
    n = ch * h * w
    xf = x.reshape(b, n)
    mf = mean_fields.reshape(s, n)
    i0 = means_idx_0.astype(jnp.int32)
    p0 = prop_means_idx_0.astype(jnp.float32)
    rate = jnp.float32(1.0 - _LAM)
    a_ev = jnp.stack([p0 * rate, (1.0 - p0) * rate], axis=1).reshape(1, 2 * b)
    s_ev = jnp.stack([i0, i0 + 1], axis=1).reshape(1, 2 * b)

    grid = (n // _BLOCK,)
    out = pl.pallas_call(
        _combine_kernel,
        grid=grid,
        in_specs=[
            pl.BlockSpec((1, _E), lambda j: (0, 0)),
            pl.BlockSpec((1, _E), lambda j: (0, 0)),
            pl.BlockSpec((s, _BLOCK), lambda j: (0, j)),
            pl.BlockSpec((b, _BLOCK), lambda j: (0, j)),
        ],
        out_specs=pl.BlockSpec((s, _BLOCK), lambda j: (0, j)),
        out_shape=jax.ShapeDtypeStruct((s, n), jnp.float32),
        scratch_shapes=[
            pltpu.VMEM((_S, _B), jnp.float32),
            pltpu.VMEM((_S, 1), jnp.float32),
        ],
    )(s_ev, a_ev, mf, xf)
    return out.reshape(s, ch, h, w)
